# baseline (speedup 1.0000x reference)
import functools

import jax
import jax.numpy as jnp
from jax import lax
from jax.experimental import pallas as pl
from jax.experimental.pallas import tpu as pltpu
from jax.experimental.pallas import tpu_sc as plsc

NC = 2
NS = 16
LANES = 16
NW = NC * NS


@functools.partial(jax.jit, static_argnums=(3, 4, 5))
def _build_and_run(z_0, z_1, eidx, E, D, chunk):
    n_per_w = E // NW
    n_chunks = n_per_w // chunk
    n_pairs = (n_chunks - 1) // 2
    groups = chunk // LANES
    mesh = plsc.VectorSubcoreMesh(core_axis_name="c", subcore_axis_name="s")

    @functools.partial(
        pl.kernel,
        out_type=jax.ShapeDtypeStruct((E,), jnp.float32),
        mesh=mesh,
        scratch_types=[
            pltpu.VMEM((n_per_w,), jnp.int32),
            pltpu.VMEM((n_per_w,), jnp.int32),
            pltpu.VMEM((chunk, D), jnp.float32),
            pltpu.VMEM((chunk, D), jnp.float32),
            pltpu.VMEM((chunk, D), jnp.float32),
            pltpu.VMEM((chunk, D), jnp.float32),
            pltpu.VMEM((n_per_w,), jnp.float32),
            pltpu.SemaphoreType.DMA,
            pltpu.SemaphoreType.DMA,
        ],
        compiler_params=pltpu.CompilerParams(needs_layout_passes=False),
    )
    def k(z0_hbm, z1_hbm, idx0_hbm, idx1_hbm, out_hbm,
          idx0_v, idx1_v, src0_v, dst0_v, src1_v, dst1_v, out_v, sem0, sem1):
        wid = lax.axis_index("s") * NC + lax.axis_index("c")
        base = wid * n_per_w
        pltpu.sync_copy(idx0_hbm.at[pl.ds(base, n_per_w)], idx0_v)
        pltpu.sync_copy(idx1_hbm.at[pl.ds(base, n_per_w)], idx1_v)

        lane_iota = jnp.arange(LANES, dtype=jnp.int32)
        zero16 = jnp.zeros((LANES,), jnp.float32)

        def gathers(kk, src_buf, dst_buf, sem):
            off = kk * chunk
            return (
                pltpu.make_async_copy(
                    z0_hbm.at[idx0_v.at[pl.ds(off, chunk)]], src_buf, sem),
                pltpu.make_async_copy(
                    z1_hbm.at[idx1_v.at[pl.ds(off, chunk)]], dst_buf, sem),
            )

        def issue(kk, src_buf, dst_buf, sem):
            for cp in gathers(kk, src_buf, dst_buf, sem):
                cp.start()

        def drain(kk, src_buf, dst_buf, sem):
            for cp in gathers(kk, src_buf, dst_buf, sem):
                cp.wait()

        cvecs = [jnp.full((LANES,), d, jnp.int32) for d in range(D)]

        def compute(kk, src_buf, dst_buf):
            off = kk * chunk

            def gbody(g, carry):
                lanes = g * LANES + lane_iota
                accs = [zero16] * 8
                for d in range(D):
                    accs[d % 8] = accs[d % 8] + (
                        plsc.load_gather(src_buf, [lanes, cvecs[d]])
                        * plsc.load_gather(dst_buf, [lanes, cvecs[d]]))
                x = (((accs[0] + accs[1]) + (accs[2] + accs[3]))
                     + ((accs[4] + accs[5]) + (accs[6] + accs[7])))
                y = 1.0 / (1.0 + jnp.exp(-x))
                out_v[pl.ds(off + g * LANES, LANES)] = y
                return carry

            lax.fori_loop(0, groups, gbody, 0)

        issue(0, src0_v, dst0_v, sem0)

        def pair_body(m, carry):
            k0 = 2 * m
            drain(k0, src0_v, dst0_v, sem0)
            issue(k0 + 1, src1_v, dst1_v, sem1)
            compute(k0, src0_v, dst0_v)
            drain(k0 + 1, src1_v, dst1_v, sem1)
            issue(k0 + 2, src0_v, dst0_v, sem0)
            compute(k0 + 1, src1_v, dst1_v)
            return carry

        lax.fori_loop(0, n_pairs, pair_body, 0)

        klast = 2 * n_pairs
        drain(klast, src0_v, dst0_v, sem0)
        if klast + 1 < n_chunks:
            issue(klast + 1, src1_v, dst1_v, sem1)
        compute(klast, src0_v, dst0_v)
        if klast + 1 < n_chunks:
            drain(klast + 1, src1_v, dst1_v, sem1)
            compute(klast + 1, src1_v, dst1_v)

        pltpu.sync_copy(out_v, out_hbm.at[pl.ds(base, n_per_w)])

    return k(z_0, z_1, eidx[0], eidx[1])


def kernel(z_0, z_1, edge_index):
    E = edge_index.shape[1]
    D = z_0.shape[1]
    eidx = edge_index.astype(jnp.int32)
    return _build_and_run(z_0, z_1, eidx, E, D, 80)


# speedup vs baseline: 1.1399x; 1.1399x over previous
import functools

import jax
import jax.numpy as jnp
from jax import lax
from jax.experimental import pallas as pl
from jax.experimental.pallas import tpu as pltpu
from jax.experimental.pallas import tpu_sc as plsc

NC = 2
NS = 16
LANES = 16
NW = NC * NS


@functools.partial(jax.jit, static_argnums=(3, 4, 5))
def _build_and_run(z_0, z_1, eidx, E, D, chunk):
    n_per_w = E // NW
    n_chunks = n_per_w // chunk
    n_pairs = (n_chunks - 1) // 2
    groups = chunk // LANES
    mesh = plsc.VectorSubcoreMesh(core_axis_name="c", subcore_axis_name="s")

    @functools.partial(
        pl.kernel,
        out_type=jax.ShapeDtypeStruct((E,), jnp.float32),
        mesh=mesh,
        scratch_types=[
            pltpu.VMEM((n_per_w,), jnp.int32),
            pltpu.VMEM((n_per_w,), jnp.int32),
            pltpu.VMEM((chunk, D), jnp.float32),
            pltpu.VMEM((chunk, D), jnp.float32),
            pltpu.VMEM((chunk, D), jnp.float32),
            pltpu.VMEM((chunk, D), jnp.float32),
            pltpu.VMEM((n_per_w,), jnp.float32),
            pltpu.SemaphoreType.DMA,
            pltpu.SemaphoreType.DMA,
        ],
        compiler_params=pltpu.CompilerParams(needs_layout_passes=False),
    )
    def k(z0_hbm, z1_hbm, idx0_hbm, idx1_hbm, out_hbm,
          idx0_v, idx1_v, src0_v, dst0_v, src1_v, dst1_v, out_v, sem0, sem1):
        wid = lax.axis_index("s") * NC + lax.axis_index("c")
        base = wid * n_per_w
        pltpu.sync_copy(idx0_hbm.at[pl.ds(base, n_per_w)], idx0_v)
        pltpu.sync_copy(idx1_hbm.at[pl.ds(base, n_per_w)], idx1_v)

        lane_iota = jnp.arange(LANES, dtype=jnp.int32)
        zero16 = jnp.zeros((LANES,), jnp.float32)

        def gathers(kk, src_buf, dst_buf, sem):
            off = kk * chunk
            return (
                pltpu.make_async_copy(
                    z0_hbm.at[idx0_v.at[pl.ds(off, chunk)]], src_buf, sem),
                pltpu.make_async_copy(
                    z1_hbm.at[idx1_v.at[pl.ds(off, chunk)]], dst_buf, sem),
            )

        def issue(kk, src_buf, dst_buf, sem):
            for cp in gathers(kk, src_buf, dst_buf, sem):
                cp.start()

        def drain(kk, src_buf, dst_buf, sem):
            for cp in gathers(kk, src_buf, dst_buf, sem):
                cp.wait()

        cbase = [jnp.full((LANES,), j, jnp.int32) for j in range(4)]

        def compute(kk, src_buf, dst_buf):
            off = kk * chunk

            def gbody(g, carry):
                lanes = g * LANES + lane_iota

                def dbody(i, accs):
                    a0, a1, a2, a3 = accs
                    d0 = i * 4
                    c0 = cbase[0] + d0
                    c1 = cbase[1] + d0
                    c2 = cbase[2] + d0
                    c3 = cbase[3] + d0
                    a0 = a0 + (plsc.load_gather(src_buf, [lanes, c0])
                               * plsc.load_gather(dst_buf, [lanes, c0]))
                    a1 = a1 + (plsc.load_gather(src_buf, [lanes, c1])
                               * plsc.load_gather(dst_buf, [lanes, c1]))
                    a2 = a2 + (plsc.load_gather(src_buf, [lanes, c2])
                               * plsc.load_gather(dst_buf, [lanes, c2]))
                    a3 = a3 + (plsc.load_gather(src_buf, [lanes, c3])
                               * plsc.load_gather(dst_buf, [lanes, c3]))
                    return a0, a1, a2, a3

                a0, a1, a2, a3 = lax.fori_loop(
                    0, D // 4, dbody, (zero16, zero16, zero16, zero16))
                x = (a0 + a1) + (a2 + a3)
                y = 1.0 / (1.0 + jnp.exp(-x))
                out_v[pl.ds(off + g * LANES, LANES)] = y
                return carry

            lax.fori_loop(0, groups, gbody, 0)

        issue(0, src0_v, dst0_v, sem0)

        def pair_body(m, carry):
            k0 = 2 * m
            drain(k0, src0_v, dst0_v, sem0)
            issue(k0 + 1, src1_v, dst1_v, sem1)
            compute(k0, src0_v, dst0_v)
            drain(k0 + 1, src1_v, dst1_v, sem1)
            issue(k0 + 2, src0_v, dst0_v, sem0)
            compute(k0 + 1, src1_v, dst1_v)
            return carry

        lax.fori_loop(0, n_pairs, pair_body, 0)

        klast = 2 * n_pairs
        drain(klast, src0_v, dst0_v, sem0)
        if klast + 1 < n_chunks:
            issue(klast + 1, src1_v, dst1_v, sem1)
        compute(klast, src0_v, dst0_v)
        if klast + 1 < n_chunks:
            drain(klast + 1, src1_v, dst1_v, sem1)
            compute(klast + 1, src1_v, dst1_v)

        pltpu.sync_copy(out_v, out_hbm.at[pl.ds(base, n_per_w)])

    return k(z_0, z_1, eidx[0], eidx[1])


def kernel(z_0, z_1, edge_index):
    E = edge_index.shape[1]
    D = z_0.shape[1]
    eidx = edge_index.astype(jnp.int32)
    return _build_and_run(z_0, z_1, eidx, E, D, 80)


# speedup vs baseline: 3.8251x; 3.3557x over previous
import functools

import jax
import jax.numpy as jnp
from jax import lax
from jax.experimental import pallas as pl
from jax.experimental.pallas import tpu as pltpu
from jax.experimental.pallas import tpu_sc as plsc

NC = 2
NS = 16
LANES = 16
NW = NC * NS


@functools.partial(jax.jit, static_argnums=(3, 4, 5))
def _build_and_run(z_0, z_1, eidx, E, D, chunk):
    n_per_w = E // NW
    n_chunks = n_per_w // chunk
    n_pairs = (n_chunks - 1) // 2
    groups = chunk // LANES
    mesh = plsc.VectorSubcoreMesh(core_axis_name="c", subcore_axis_name="s")

    @functools.partial(
        pl.kernel,
        out_type=jax.ShapeDtypeStruct((E,), jnp.float32),
        mesh=mesh,
        scratch_types=[
            pltpu.VMEM((n_per_w,), jnp.int32),
            pltpu.VMEM((n_per_w,), jnp.int32),
            pltpu.VMEM((chunk, D), jnp.float32),
            pltpu.VMEM((chunk, D), jnp.float32),
            pltpu.VMEM((chunk, D), jnp.float32),
            pltpu.VMEM((chunk, D), jnp.float32),
            pltpu.VMEM((n_per_w,), jnp.float32),
            pltpu.SemaphoreType.DMA,
            pltpu.SemaphoreType.DMA,
        ],
        compiler_params=pltpu.CompilerParams(needs_layout_passes=False),
    )
    def k(z0_hbm, z1_hbm, idx0_hbm, idx1_hbm, out_hbm,
          idx0_v, idx1_v, src0_v, dst0_v, src1_v, dst1_v, out_v, sem0, sem1):
        wid = lax.axis_index("s") * NC + lax.axis_index("c")
        base = wid * n_per_w
        pltpu.sync_copy(idx0_hbm.at[pl.ds(base, n_per_w)], idx0_v)
        pltpu.sync_copy(idx1_hbm.at[pl.ds(base, n_per_w)], idx1_v)

        lane_iota = jnp.arange(LANES, dtype=jnp.int32)
        zero16 = jnp.zeros((LANES,), jnp.float32)

        def gathers(kk, src_buf, dst_buf, sem):
            off = kk * chunk
            return (
                pltpu.make_async_copy(
                    z0_hbm.at[idx0_v.at[pl.ds(off, chunk)]], src_buf, sem),
                pltpu.make_async_copy(
                    z1_hbm.at[idx1_v.at[pl.ds(off, chunk)]], dst_buf, sem),
            )

        def issue(kk, src_buf, dst_buf, sem):
            for cp in gathers(kk, src_buf, dst_buf, sem):
                cp.start()

        def drain(kk, src_buf, dst_buf, sem):
            for cp in gathers(kk, src_buf, dst_buf, sem):
                cp.wait()

        def compute(kk, src_buf, dst_buf):
            off = kk * chunk

            def gbody(g, carry):
                base_e = g * LANES
                r = zero16
                for j in range(LANES):
                    e = base_e + j
                    ps = [src_buf[e, pl.ds(t * LANES, LANES)]
                          * dst_buf[e, pl.ds(t * LANES, LANES)]
                          for t in range(D // LANES)]
                    while len(ps) > 1:
                        ps = [ps[i] + ps[i + 1] for i in range(0, len(ps), 2)]
                    s = jnp.sum(ps[0])
                    r = jnp.where(lane_iota == j, s, r)
                y = 1.0 / (1.0 + jnp.exp(-r))
                out_v[pl.ds(off + base_e, LANES)] = y
                return carry

            lax.fori_loop(0, groups, gbody, 0)

        issue(0, src0_v, dst0_v, sem0)

        def pair_body(m, carry):
            k0 = 2 * m
            drain(k0, src0_v, dst0_v, sem0)
            issue(k0 + 1, src1_v, dst1_v, sem1)
            compute(k0, src0_v, dst0_v)
            drain(k0 + 1, src1_v, dst1_v, sem1)
            issue(k0 + 2, src0_v, dst0_v, sem0)
            compute(k0 + 1, src1_v, dst1_v)
            return carry

        lax.fori_loop(0, n_pairs, pair_body, 0)

        klast = 2 * n_pairs
        drain(klast, src0_v, dst0_v, sem0)
        if klast + 1 < n_chunks:
            issue(klast + 1, src1_v, dst1_v, sem1)
        compute(klast, src0_v, dst0_v)
        if klast + 1 < n_chunks:
            drain(klast + 1, src1_v, dst1_v, sem1)
            compute(klast + 1, src1_v, dst1_v)

        pltpu.sync_copy(out_v, out_hbm.at[pl.ds(base, n_per_w)])

    return k(z_0, z_1, eidx[0], eidx[1])


def kernel(z_0, z_1, edge_index):
    E = edge_index.shape[1]
    D = z_0.shape[1]
    eidx = edge_index.astype(jnp.int32)
    return _build_and_run(z_0, z_1, eidx, E, D, 80)


# speedup vs baseline: 6.3047x; 1.6483x over previous
import functools

import jax
import jax.numpy as jnp
from jax import lax
from jax.experimental import pallas as pl
from jax.experimental.pallas import tpu as pltpu
from jax.experimental.pallas import tpu_sc as plsc

NC = 2
NS = 16
LANES = 16
NW = NC * NS


@functools.partial(jax.jit, static_argnums=(3, 4, 5))
def _build_and_run(z_0, z_1, eidx, E, D, chunk):
    n_per_w = E // NW
    n_chunks = n_per_w // chunk
    n_pairs = (n_chunks - 1) // 2
    groups = chunk // LANES
    mesh = plsc.VectorSubcoreMesh(core_axis_name="c", subcore_axis_name="s")

    @functools.partial(
        pl.kernel,
        out_type=jax.ShapeDtypeStruct((E,), jnp.float32),
        mesh=mesh,
        scratch_types=[
            pltpu.VMEM((n_per_w,), jnp.int32),
            pltpu.VMEM((n_per_w,), jnp.int32),
            pltpu.VMEM((chunk, D), jnp.float32),
            pltpu.VMEM((chunk, D), jnp.float32),
            pltpu.VMEM((chunk, D), jnp.float32),
            pltpu.VMEM((chunk, D), jnp.float32),
            pltpu.VMEM((n_per_w,), jnp.float32),
            pltpu.SemaphoreType.DMA,
            pltpu.SemaphoreType.DMA,
        ],
        compiler_params=pltpu.CompilerParams(needs_layout_passes=False),
    )
    def k(z0_hbm, z1_hbm, idx0_hbm, idx1_hbm, out_hbm,
          idx0_v, idx1_v, src0_v, dst0_v, src1_v, dst1_v, out_v, sem0, sem1):
        wid = lax.axis_index("s") * NC + lax.axis_index("c")
        base = wid * n_per_w
        pltpu.sync_copy(idx0_hbm.at[pl.ds(base, n_per_w)], idx0_v)
        pltpu.sync_copy(idx1_hbm.at[pl.ds(base, n_per_w)], idx1_v)

        lane_iota = jnp.arange(LANES, dtype=jnp.int32)
        zero16 = jnp.zeros((LANES,), jnp.float32)

        def gathers(kk, src_buf, dst_buf, sem):
            off = kk * chunk
            return (
                pltpu.make_async_copy(
                    z0_hbm.at[idx0_v.at[pl.ds(off, chunk)]], src_buf, sem),
                pltpu.make_async_copy(
                    z1_hbm.at[idx1_v.at[pl.ds(off, chunk)]], dst_buf, sem),
            )

        def issue(kk, src_buf, dst_buf, sem):
            for cp in gathers(kk, src_buf, dst_buf, sem):
                cp.start()

        def drain(kk, src_buf, dst_buf, sem):
            for cp in gathers(kk, src_buf, dst_buf, sem):
                cp.wait()

        def compute(kk, src_buf, dst_buf):
            off = kk * chunk

            def gbody(g, carry):
                lanes = g * LANES + lane_iota
                def dbody(i, accs):
                    a0, a1, a2, a3 = accs
                    d0 = i * 4
                    c0 = (lane_iota + d0) & (D - 1)
                    c1 = (lane_iota + (d0 + 1)) & (D - 1)
                    c2 = (lane_iota + (d0 + 2)) & (D - 1)
                    c3 = (lane_iota + (d0 + 3)) & (D - 1)
                    a0 = a0 + (plsc.load_gather(src_buf, [lanes, c0])
                               * plsc.load_gather(dst_buf, [lanes, c0]))
                    a1 = a1 + (plsc.load_gather(src_buf, [lanes, c1])
                               * plsc.load_gather(dst_buf, [lanes, c1]))
                    a2 = a2 + (plsc.load_gather(src_buf, [lanes, c2])
                               * plsc.load_gather(dst_buf, [lanes, c2]))
                    a3 = a3 + (plsc.load_gather(src_buf, [lanes, c3])
                               * plsc.load_gather(dst_buf, [lanes, c3]))
                    return a0, a1, a2, a3

                a0, a1, a2, a3 = lax.fori_loop(
                    0, D // 4, dbody, (zero16, zero16, zero16, zero16))
                x = (a0 + a1) + (a2 + a3)
                y = 1.0 / (1.0 + jnp.exp(-x))
                out_v[pl.ds(off + g * LANES, LANES)] = y
                return carry

            lax.fori_loop(0, groups, gbody, 0)

        issue(0, src0_v, dst0_v, sem0)

        def pair_body(m, carry):
            k0 = 2 * m
            drain(k0, src0_v, dst0_v, sem0)
            issue(k0 + 1, src1_v, dst1_v, sem1)
            compute(k0, src0_v, dst0_v)
            drain(k0 + 1, src1_v, dst1_v, sem1)
            issue(k0 + 2, src0_v, dst0_v, sem0)
            compute(k0 + 1, src1_v, dst1_v)
            return carry

        lax.fori_loop(0, n_pairs, pair_body, 0)

        klast = 2 * n_pairs
        drain(klast, src0_v, dst0_v, sem0)
        if klast + 1 < n_chunks:
            issue(klast + 1, src1_v, dst1_v, sem1)
        compute(klast, src0_v, dst0_v)
        if klast + 1 < n_chunks:
            drain(klast + 1, src1_v, dst1_v, sem1)
            compute(klast + 1, src1_v, dst1_v)

        pltpu.sync_copy(out_v, out_hbm.at[pl.ds(base, n_per_w)])

    return k(z_0, z_1, eidx[0], eidx[1])


def kernel(z_0, z_1, edge_index):
    E = edge_index.shape[1]
    D = z_0.shape[1]
    eidx = edge_index.astype(jnp.int32)
    return _build_and_run(z_0, z_1, eidx, E, D, 80)


# speedup vs baseline: 9.7941x; 1.5535x over previous
import functools

import jax
import jax.numpy as jnp
from jax import lax
from jax.experimental import pallas as pl
from jax.experimental.pallas import tpu as pltpu
from jax.experimental.pallas import tpu_sc as plsc

NC = 2
NS = 16
LANES = 16
NW = NC * NS
NBUF = 4


@functools.partial(jax.jit, static_argnums=(3, 4, 5))
def _build_and_run(z_0, z_1, eidx, E, D, chunk):
    n_per_w = E // NW
    n_chunks = n_per_w // chunk
    n_quads = n_chunks // NBUF
    n_tail = n_chunks - NBUF * n_quads
    groups = chunk // LANES
    mesh = plsc.VectorSubcoreMesh(core_axis_name="c", subcore_axis_name="s")

    row_bufs = []
    for _ in range(NBUF):
        row_bufs += [pltpu.VMEM((chunk, D), jnp.float32),
                     pltpu.VMEM((chunk, D), jnp.float32)]

    @functools.partial(
        pl.kernel,
        out_type=jax.ShapeDtypeStruct((E,), jnp.float32),
        mesh=mesh,
        scratch_types=[
            pltpu.VMEM((n_per_w,), jnp.int32),
            pltpu.VMEM((n_per_w,), jnp.int32),
            *row_bufs,
            pltpu.VMEM((n_per_w,), jnp.float32),
            *([pltpu.SemaphoreType.DMA] * NBUF),
        ],
        compiler_params=pltpu.CompilerParams(needs_layout_passes=False),
    )
    def k(z0_hbm, z1_hbm, idx0_hbm, idx1_hbm, out_hbm,
          idx0_v, idx1_v, *rest):
        bufs = [(rest[2 * b], rest[2 * b + 1]) for b in range(NBUF)]
        out_v = rest[2 * NBUF]
        sems = rest[2 * NBUF + 1:]

        wid = lax.axis_index("s") * NC + lax.axis_index("c")
        base = wid * n_per_w
        pltpu.sync_copy(idx0_hbm.at[pl.ds(base, n_per_w)], idx0_v)
        pltpu.sync_copy(idx1_hbm.at[pl.ds(base, n_per_w)], idx1_v)

        lane_iota = jnp.arange(LANES, dtype=jnp.int32)
        zero16 = jnp.zeros((LANES,), jnp.float32)

        def gathers(kk, b):
            off = kk * chunk
            src_buf, dst_buf = bufs[b]
            return (
                pltpu.make_async_copy(
                    z0_hbm.at[idx0_v.at[pl.ds(off, chunk)]], src_buf, sems[b]),
                pltpu.make_async_copy(
                    z1_hbm.at[idx1_v.at[pl.ds(off, chunk)]], dst_buf, sems[b]),
            )

        def issue(kk, b):
            for cp in gathers(kk, b):
                cp.start()

        def drain(kk, b):
            for cp in gathers(kk, b):
                cp.wait()

        def compute(kk, b):
            off = kk * chunk
            src_buf, dst_buf = bufs[b]

            def gbody(g, carry):
                lanes = g * LANES + lane_iota

                def dbody(i, accs):
                    a0, a1, a2, a3 = accs
                    d0 = i * 4
                    c0 = (lane_iota + d0) & (D - 1)
                    c1 = (lane_iota + (d0 + 1)) & (D - 1)
                    c2 = (lane_iota + (d0 + 2)) & (D - 1)
                    c3 = (lane_iota + (d0 + 3)) & (D - 1)
                    a0 = a0 + (plsc.load_gather(src_buf, [lanes, c0])
                               * plsc.load_gather(dst_buf, [lanes, c0]))
                    a1 = a1 + (plsc.load_gather(src_buf, [lanes, c1])
                               * plsc.load_gather(dst_buf, [lanes, c1]))
                    a2 = a2 + (plsc.load_gather(src_buf, [lanes, c2])
                               * plsc.load_gather(dst_buf, [lanes, c2]))
                    a3 = a3 + (plsc.load_gather(src_buf, [lanes, c3])
                               * plsc.load_gather(dst_buf, [lanes, c3]))
                    return a0, a1, a2, a3

                a0, a1, a2, a3 = lax.fori_loop(
                    0, D // 4, dbody, (zero16, zero16, zero16, zero16))
                x = (a0 + a1) + (a2 + a3)
                y = 1.0 / (1.0 + jnp.exp(-x))
                out_v[pl.ds(off + g * LANES, LANES)] = y
                return carry

            lax.fori_loop(0, groups, gbody, 0)

        for b in range(NBUF - 1):
            issue(b, b)

        def quad_body(m, carry):
            for b in range(NBUF):
                kk = NBUF * m + b
                drain(kk, b)

                @pl.when(kk + NBUF - 1 < n_chunks)
                def _():
                    issue(kk + NBUF - 1, (b + NBUF - 1) % NBUF)

                compute(kk, b)
            return carry

        lax.fori_loop(0, n_quads, quad_body, 0)

        for t in range(n_tail):
            kk = NBUF * n_quads + t
            drain(kk, kk % NBUF)
            compute(kk, kk % NBUF)

        pltpu.sync_copy(out_v, out_hbm.at[pl.ds(base, n_per_w)])

    return k(z_0, z_1, eidx[0], eidx[1])


def kernel(z_0, z_1, edge_index):
    E = edge_index.shape[1]
    D = z_0.shape[1]
    eidx = edge_index.astype(jnp.int32)
    return _build_and_run(z_0, z_1, eidx, E, D, 80)
